# baseline (speedup 1.0000x reference)
import jax
import jax.numpy as jnp
from jax.experimental import pallas as pl
from jax.experimental.pallas import tpu as pltpu


def _body(h_ref, w_ref, adj_ref, out_ref, hw_scr):
    i = pl.program_id(0)

    @pl.when(i == 0)
    def _hw():
        hw_scr[...] = jnp.dot(
            h_ref[...].astype(jnp.bfloat16),
            w_ref[...].astype(jnp.bfloat16),
            preferred_element_type=jnp.float32,
        ).astype(jnp.bfloat16)

    @pl.when(i > 0)
    def _mm():
        out_ref[...] = jnp.dot(
            adj_ref[...].astype(jnp.bfloat16),
            hw_scr[...],
            preferred_element_type=jnp.float32,
        )


def kernel(h, adj, W):
    n, in_f = h.shape
    out_f = W.shape[1]
    br = 80

    def _adj_idx(i):
        return (jnp.maximum(i - 1, 0), 0)

    out = pl.pallas_call(
        _body,
        grid=(n // br + 1,),
        in_specs=[
            pl.BlockSpec((n, in_f), lambda i: (0, 0)),
            pl.BlockSpec((in_f, out_f), lambda i: (0, 0)),
            pl.BlockSpec((br, n), _adj_idx),
        ],
        out_specs=pl.BlockSpec((br, out_f), _adj_idx),
        out_shape=jax.ShapeDtypeStruct((n, out_f), jnp.float32),
        scratch_shapes=[pltpu.VMEM((n, out_f), jnp.bfloat16)],
        compiler_params=pltpu.CompilerParams(
            dimension_semantics=("arbitrary",),
        ),
    )(h, W, adj)
    return out


# speedup vs baseline: 1.3741x; 1.3741x over previous
import jax
import jax.numpy as jnp
from jax.experimental import pallas as pl
from jax.experimental.pallas import tpu as pltpu


def _body(h_ref, w_ref, adj_ref, out_ref, hw_scr):
    i = pl.program_id(0)

    @pl.when(i == 0)
    def _hw():
        hw_scr[...] = jnp.dot(
            h_ref[...].astype(jnp.bfloat16),
            w_ref[...].astype(jnp.bfloat16),
            preferred_element_type=jnp.float32,
        ).astype(jnp.bfloat16)

    @pl.when(i > 0)
    def _mm():
        out_ref[...] = jnp.dot(
            adj_ref[...].astype(jnp.bfloat16),
            hw_scr[...],
            preferred_element_type=jnp.float32,
        )


def kernel(h, adj, W):
    n, in_f = h.shape
    out_f = W.shape[1]
    br = 400

    def _adj_idx(i):
        return (jnp.maximum(i - 1, 0), 0)

    out = pl.pallas_call(
        _body,
        grid=(n // br + 1,),
        in_specs=[
            pl.BlockSpec((n, in_f), lambda i: (0, 0)),
            pl.BlockSpec((in_f, out_f), lambda i: (0, 0)),
            pl.BlockSpec((br, n), _adj_idx),
        ],
        out_specs=pl.BlockSpec((br, out_f), _adj_idx),
        out_shape=jax.ShapeDtypeStruct((n, out_f), jnp.float32),
        scratch_shapes=[pltpu.VMEM((n, out_f), jnp.bfloat16)],
        compiler_params=pltpu.CompilerParams(
            dimension_semantics=("arbitrary",),
        ),
    )(h, W, adj)
    return out


# speedup vs baseline: 1.3991x; 1.0182x over previous
import jax
import jax.numpy as jnp
from jax.experimental import pallas as pl
from jax.experimental.pallas import tpu as pltpu


def _body(h_ref, w_ref, adj_ref, out_ref, hw_scr):
    i = pl.program_id(0)

    @pl.when(i == 0)
    def _hw():
        hw_scr[...] = jnp.dot(
            h_ref[...].astype(jnp.bfloat16),
            w_ref[...].astype(jnp.bfloat16),
            preferred_element_type=jnp.float32,
        ).astype(jnp.bfloat16)

    out_ref[...] = jnp.dot(
        adj_ref[...].astype(jnp.bfloat16),
        hw_scr[...],
        preferred_element_type=jnp.float32,
    )


def kernel(h, adj, W):
    n, in_f = h.shape
    out_f = W.shape[1]
    br = 400

    out = pl.pallas_call(
        _body,
        grid=(n // br,),
        in_specs=[
            pl.BlockSpec((n, in_f), lambda i: (0, 0)),
            pl.BlockSpec((in_f, out_f), lambda i: (0, 0)),
            pl.BlockSpec((br, n), lambda i: (i, 0)),
        ],
        out_specs=pl.BlockSpec((br, out_f), lambda i: (i, 0)),
        out_shape=jax.ShapeDtypeStruct((n, out_f), jnp.float32),
        scratch_shapes=[pltpu.VMEM((n, out_f), jnp.bfloat16)],
        compiler_params=pltpu.CompilerParams(
            dimension_semantics=("arbitrary",),
        ),
    )(h, W, adj)
    return out
